# baseline (speedup 1.0000x reference)
import math

import jax
import jax.numpy as jnp
from jax import lax
from jax.experimental import pallas as pl
from jax.experimental.pallas import tpu as pltpu
from jax.experimental.pallas import tpu_sc as plsc

B = 64
T = 512
D = 1024
PAD = 1
SCALE = math.sqrt(D)
NW = 32
TCH = T // NW
NL = 16
NCH = D // NL

NEG_INF = float("-inf")


def _body(x_hbm, y_hbm, w_hbm, pos_hbm,
          xe_hbm, ms_hbm, ye_hbm, yt_hbm, mt_hbm,
          pos_v, idx_x, idx_y, rows_x, rows_y, msk_x, msk_y,
          semi, semg_x, semg_y, sems_x, sems_y):
    c = lax.axis_index("c")
    s = lax.axis_index("s")
    wid = s * 2 + c
    t0 = pl.multiple_of(wid * TCH, TCH)
    last = wid == NW - 1

    def fire_ids(b, carry):
        off = pl.multiple_of(b * T + t0, TCH)
        pltpu.async_copy(x_hbm.at[pl.ds(off, TCH)], idx_x.at[b], semi)
        pltpu.async_copy(y_hbm.at[pl.ds(off, TCH)], idx_y.at[b], semi)
        return carry

    lax.fori_loop(0, B, fire_ids, 0)

    pltpu.sync_copy(pos_hbm.at[pl.ds(t0, TCH)], pos_v)

    def drain_ids(b, carry):
        pltpu.make_async_copy(x_hbm.at[pl.ds(0, TCH)], idx_x.at[b],
                              semi).wait()
        pltpu.make_async_copy(y_hbm.at[pl.ds(0, TCH)], idx_y.at[b],
                              semi).wait()
        return carry

    lax.fori_loop(0, B, drain_ids, 0)

    def fire(b, p):
        pltpu.async_copy(w_hbm.at[idx_x.at[b]], rows_x.at[p], semg_x.at[p])
        pltpu.async_copy(w_hbm.at[idx_y.at[b]], rows_y.at[p], semg_y.at[p])

    def wait_gather(p):
        pltpu.make_async_copy(w_hbm.at[idx_x.at[0]], rows_x.at[p],
                              semg_x.at[p]).wait()
        pltpu.make_async_copy(w_hbm.at[idx_y.at[0]], rows_y.at[p],
                              semg_y.at[p]).wait()

    def store(b, p):
        pltpu.async_copy(rows_x.at[p], xe_hbm.at[b, pl.ds(t0, TCH)],
                         sems_x.at[p])

        @pl.when(jnp.logical_not(last))
        def _():
            pltpu.async_copy(rows_y.at[p], ye_hbm.at[b, pl.ds(t0, TCH)],
                             sems_y.at[p])

        @pl.when(last)
        def _():
            pltpu.async_copy(rows_y.at[p], yt_hbm.at[b], sems_y.at[p])

    def wait_store(p):
        pltpu.make_async_copy(rows_x.at[p], xe_hbm.at[0, pl.ds(t0, TCH)],
                              sems_x.at[p]).wait()

        @pl.when(jnp.logical_not(last))
        def _():
            pltpu.make_async_copy(rows_y.at[p],
                                  ye_hbm.at[0, pl.ds(t0, TCH)],
                                  sems_y.at[p]).wait()

        @pl.when(last)
        def _():
            pltpu.make_async_copy(rows_y.at[p], yt_hbm.at[0],
                                  sems_y.at[p]).wait()

    fire(0, 0)

    def batch_step(b, carry):
        p = b & 1
        q = 1 - p

        @pl.when(b < B - 1)
        def _():
            @pl.when(b >= 1)
            def _():
                wait_store(q)

            fire(b + 1, q)

        wait_gather(p)

        vx = idx_x[b, :]
        vy = idx_y[b, :]
        msk_x[b, :] = jnp.where(vx == PAD, NEG_INF, 0.0)
        msk_y[b, :] = jnp.where(vy == PAD, NEG_INF, 0.0)

        @plsc.parallel_loop(0, TCH * NCH, 1, unroll=8)
        def compute(i):
            r = i >> 6
            o = (i & (NCH - 1)) * NL
            pv = pos_v[r, pl.ds(o, NL)]
            rows_x[p, r, pl.ds(o, NL)] = rows_x[p, r, pl.ds(o, NL)] * SCALE + pv
            rows_y[p, r, pl.ds(o, NL)] = rows_y[p, r, pl.ds(o, NL)] * SCALE + pv

        store(b, p)
        return carry

    lax.fori_loop(0, B, batch_step, 0)
    wait_store(0)
    wait_store(1)

    pltpu.sync_copy(msk_x, ms_hbm.at[wid])
    pltpu.sync_copy(msk_y, mt_hbm.at[wid])


@jax.jit
def kernel(x, y, W, pos_w):
    mesh = plsc.VectorSubcoreMesh(core_axis_name="c", subcore_axis_name="s")
    f = pl.kernel(
        _body,
        out_type=[
            jax.ShapeDtypeStruct((B, T, D), jnp.float32),
            jax.ShapeDtypeStruct((NW, B, TCH), jnp.float32),
            jax.ShapeDtypeStruct((B, T - 1, D), jnp.float32),
            jax.ShapeDtypeStruct((B, TCH, D), jnp.float32),
            jax.ShapeDtypeStruct((NW, B, TCH), jnp.float32),
        ],
        mesh=mesh,
        scratch_types=[
            pltpu.VMEM((TCH, D), jnp.float32),
            pltpu.VMEM((B, TCH), jnp.int32),
            pltpu.VMEM((B, TCH), jnp.int32),
            pltpu.VMEM((2, TCH, D), jnp.float32),
            pltpu.VMEM((2, TCH, D), jnp.float32),
            pltpu.VMEM((B, TCH), jnp.float32),
            pltpu.VMEM((B, TCH), jnp.float32),
            pltpu.SemaphoreType.DMA,
            pltpu.SemaphoreType.DMA((2,)),
            pltpu.SemaphoreType.DMA((2,)),
            pltpu.SemaphoreType.DMA((2,)),
            pltpu.SemaphoreType.DMA((2,)),
        ],
    )
    xf = x.reshape(-1)
    yf = y.reshape(-1)
    x_emb, msk3, ye_main, yt, mtk3 = f(xf, yf, W, pos_w)
    mask_src = msk3.transpose(1, 0, 2).reshape(B, T)
    mask_tgt = mtk3.transpose(1, 0, 2).reshape(B, T)[:, : T - 1]
    y_emb = lax.dynamic_update_slice(ye_main, yt[:, : TCH - 1],
                                     (0, T - TCH, 0))
    labels = y[:, 1:]
    return (x_emb, mask_src, mask_src, y_emb, mask_tgt, labels)


# speedup vs baseline: 1.0106x; 1.0106x over previous
import math

import jax
import jax.numpy as jnp
from jax import lax
from jax.experimental import pallas as pl
from jax.experimental.pallas import tpu as pltpu
from jax.experimental.pallas import tpu_sc as plsc

B = 64
T = 512
D = 1024
PAD = 1
SCALE = math.sqrt(D)
NW = 32
TCH = T // NW
NL = 16
NCH = D // NL

NEG_INF = float("-inf")


def _body(x_hbm, y_hbm, w_hbm, pos_hbm,
          xe_hbm, ms_hbm, ye_hbm, mt_hbm,
          pos_v, idx_x, idx_y, rows_x, rows_y, msk_x, msk_y,
          semi, semg_x, semg_y, sems_x, sems_y):
    c = lax.axis_index("c")
    s = lax.axis_index("s")
    wid = s * 2 + c
    t0 = pl.multiple_of(wid * TCH, TCH)

    def fire_ids(b, carry):
        off = pl.multiple_of(b * T + t0, TCH)
        pltpu.async_copy(x_hbm.at[pl.ds(off, TCH)], idx_x.at[b], semi)
        pltpu.async_copy(y_hbm.at[pl.ds(off, TCH)], idx_y.at[b], semi)
        return carry

    lax.fori_loop(0, B, fire_ids, 0)

    pltpu.sync_copy(pos_hbm.at[pl.ds(t0, TCH)], pos_v)

    def drain_ids(b, carry):
        pltpu.make_async_copy(x_hbm.at[pl.ds(0, TCH)], idx_x.at[b],
                              semi).wait()
        pltpu.make_async_copy(y_hbm.at[pl.ds(0, TCH)], idx_y.at[b],
                              semi).wait()
        return carry

    lax.fori_loop(0, B, drain_ids, 0)

    def fire(b, p):
        pltpu.async_copy(w_hbm.at[idx_x.at[b]], rows_x.at[p], semg_x.at[p])
        pltpu.async_copy(w_hbm.at[idx_y.at[b]], rows_y.at[p], semg_y.at[p])

    def wait_gather(p):
        pltpu.make_async_copy(w_hbm.at[idx_x.at[0]], rows_x.at[p],
                              semg_x.at[p]).wait()
        pltpu.make_async_copy(w_hbm.at[idx_y.at[0]], rows_y.at[p],
                              semg_y.at[p]).wait()

    def store(b, p):
        pltpu.async_copy(rows_x.at[p], xe_hbm.at[b, pl.ds(t0, TCH)],
                         sems_x.at[p])
        pltpu.async_copy(rows_y.at[p], ye_hbm.at[b, pl.ds(t0, TCH)],
                         sems_y.at[p])

    def wait_store(p):
        pltpu.make_async_copy(rows_x.at[p], xe_hbm.at[0, pl.ds(t0, TCH)],
                              sems_x.at[p]).wait()
        pltpu.make_async_copy(rows_y.at[p], ye_hbm.at[0, pl.ds(t0, TCH)],
                              sems_y.at[p]).wait()

    fire(0, 0)

    def batch_step(b, carry):
        p = b & 1
        q = 1 - p

        @pl.when(b < B - 1)
        def _():
            @pl.when(b >= 1)
            def _():
                wait_store(q)

            fire(b + 1, q)

        wait_gather(p)

        vx = idx_x[b, :]
        vy = idx_y[b, :]
        msk_x[b, :] = jnp.where(vx == PAD, NEG_INF, 0.0)
        msk_y[b, :] = jnp.where(vy == PAD, NEG_INF, 0.0)

        @plsc.parallel_loop(0, TCH * NCH, 1, unroll=8)
        def compute(i):
            r = i >> 6
            o = (i & (NCH - 1)) * NL
            pv = pos_v[r, pl.ds(o, NL)]
            rows_x[p, r, pl.ds(o, NL)] = rows_x[p, r, pl.ds(o, NL)] * SCALE + pv
            rows_y[p, r, pl.ds(o, NL)] = rows_y[p, r, pl.ds(o, NL)] * SCALE + pv

        store(b, p)
        return carry

    lax.fori_loop(0, B, batch_step, 0)
    wait_store(0)
    wait_store(1)

    pltpu.sync_copy(msk_x, ms_hbm.at[wid])
    pltpu.sync_copy(msk_y, mt_hbm.at[wid])


@jax.jit
def kernel(x, y, W, pos_w):
    mesh = plsc.VectorSubcoreMesh(core_axis_name="c", subcore_axis_name="s")
    f = pl.kernel(
        _body,
        out_type=[
            jax.ShapeDtypeStruct((B, T, D), jnp.float32),
            jax.ShapeDtypeStruct((NW, B, TCH), jnp.float32),
            jax.ShapeDtypeStruct((B, T, D), jnp.float32),
            jax.ShapeDtypeStruct((NW, B, TCH), jnp.float32),
        ],
        mesh=mesh,
        scratch_types=[
            pltpu.VMEM((TCH, D), jnp.float32),
            pltpu.VMEM((B, TCH), jnp.int32),
            pltpu.VMEM((B, TCH), jnp.int32),
            pltpu.VMEM((2, TCH, D), jnp.float32),
            pltpu.VMEM((2, TCH, D), jnp.float32),
            pltpu.VMEM((B, TCH), jnp.float32),
            pltpu.VMEM((B, TCH), jnp.float32),
            pltpu.SemaphoreType.DMA,
            pltpu.SemaphoreType.DMA((2,)),
            pltpu.SemaphoreType.DMA((2,)),
            pltpu.SemaphoreType.DMA((2,)),
            pltpu.SemaphoreType.DMA((2,)),
        ],
    )
    xf = x.reshape(-1)
    yf = y.reshape(-1)
    x_emb, msk3, ye_full, mtk3 = f(xf, yf, W, pos_w)
    mask_src = msk3.transpose(1, 0, 2).reshape(B, T)
    mask_tgt = mtk3.transpose(1, 0, 2).reshape(B, T)[:, : T - 1]
    y_emb = ye_full[:, : T - 1]
    labels = y[:, 1:]
    return (x_emb, mask_src, mask_src, y_emb, mask_tgt, labels)
